# baseline (speedup 1.0000x reference)
import functools

import jax
import jax.numpy as jnp
from jax import lax
from jax.experimental import pallas as pl
from jax.experimental.pallas import tpu as pltpu
from jax.experimental.pallas import tpu_sc as plsc

_VOCAB = 32002
_HIDDEN = 2048
_BATCH = 4
_SEQ = 4096
_NTOK = _BATCH * _SEQ
_NW = 32
_PER_W = _NTOK // _NW
_CHUNK = 8
_NCHUNK = _PER_W // _CHUNK
_GBUF = 2
_OBUF = 4

_mesh = plsc.VectorSubcoreMesh(core_axis_name="c", subcore_axis_name="s")


@functools.partial(
    pl.kernel,
    out_type=jax.ShapeDtypeStruct((_NTOK, _HIDDEN), jnp.float32),
    mesh=_mesh,
    scratch_types=(
        [pltpu.VMEM((_NCHUNK, _CHUNK), jnp.int32)]
        + [pltpu.VMEM((_CHUNK, _HIDDEN), jnp.float32)] * _GBUF
        + [pltpu.VMEM_SHARED((16, _OBUF, _CHUNK, _HIDDEN), jnp.float32)]
        + [pltpu.SemaphoreType.DMA] * (2 * _GBUF + _OBUF)
    ),
)
def _embed_lookup(table_hbm, idx_hbm, out_hbm, idx_v, *scratch):
    sid = lax.axis_index("s")
    wid = sid * 2 + lax.axis_index("c")
    base = wid * _PER_W
    pltpu.sync_copy(idx_hbm.at[wid], idx_v)

    bufs = scratch[:_GBUF]
    shared = scratch[_GBUF]
    gsems = scratch[_GBUF + 1:2 * _GBUF + 1]
    xsems = scratch[2 * _GBUF + 1:3 * _GBUF + 1]
    osems = scratch[3 * _GBUF + 1:]

    def gather_start(j, g):
        pltpu.async_copy(table_hbm.at[idx_v.at[j]], bufs[g], gsems[g])

    def gather_wait(g):
        pltpu.make_async_copy(
            table_hbm.at[idx_v.at[0]], bufs[g], gsems[g]).wait()

    def stage(g, o):
        pltpu.async_copy(bufs[g], shared.at[sid, o], xsems[g]).wait()

    def out_start(j, o):
        pltpu.async_copy(
            shared.at[sid, o],
            out_hbm.at[pl.ds(base + j * _CHUNK, _CHUNK)], osems[o])

    def out_wait(o):
        pltpu.make_async_copy(
            shared.at[sid, o],
            out_hbm.at[pl.ds(base, _CHUNK)], osems[o]).wait()

    def slot_body(j, g, o, skip_out_wait=False, prefetch=True):
        if not skip_out_wait:
            out_wait(o)
        gather_wait(g)
        stage(g, o)
        out_start(j, o)
        if prefetch:
            gather_start(j + _GBUF, g)

    for g in range(_GBUF):
        gather_start(g, g)
    for j in range(_OBUF):
        slot_body(j, j % _GBUF, j, skip_out_wait=True)

    def step(k, carry):
        for p in range(_OBUF):
            j = _OBUF * k + p
            slot_body(j, p % _GBUF, p)
        return carry

    lax.fori_loop(1, _NCHUNK // _OBUF - 1, step, 0)

    for p in range(_OBUF):
        j = _NCHUNK - _OBUF + p
        slot_body(j, p % _GBUF, p, prefetch=(j + _GBUF < _NCHUNK))
    for o in range(_OBUF):
        out_wait(o)


def kernel(input_ids, embedding_table):
    ids = input_ids.reshape(_NW, _NCHUNK, _CHUNK)
    out = _embed_lookup(embedding_table, ids)
    return out.reshape(_BATCH, _SEQ, _HIDDEN)
